# baseline (speedup 1.0000x reference)
import functools

import jax
import jax.numpy as jnp
from jax import lax
from jax.experimental import pallas as pl
from jax.experimental.pallas import tpu as pltpu
from jax.experimental.pallas import tpu_sc as plsc

B, N, D = 64, 32, 128
G = 8
GG = G * G
NH = 4.0
CELL = NH / G

ROWS = B * N
ROWS_PAD = ROWS + 8
ZERO_ROW = ROWS
NUM_WORKERS = 32
CHUNK = ROWS // NUM_WORKERS
GROWS = 128
NT = CHUNK * N // GROWS
CELLS_PER_BLK = 4


def _mm_body(h_ref, w_ref, o_ref):
    h = h_ref[...]
    for c in range(CELLS_PER_BLK):
        wblk = w_ref[:, c * D:(c + 1) * D]
        o_ref[c * ROWS_PAD:(c + 1) * ROWS_PAD, :] = lax.dot_general(
            h, wblk, (((1,), (1,)), ((), ())),
            preferred_element_type=jnp.float32)


def _idx_body(px_ref, py_ref, o_ref):
    px = px_ref[...]
    py = py_ref[...]
    rx = px[:, None, :] - px[:, :, None]
    ry = py[:, None, :] - py[:, :, None]
    gx = jnp.clip(((rx + NH / 2.0) / CELL).astype(jnp.int32), 0, G - 1)
    gy = jnp.clip(((ry + NH / 2.0) / CELL).astype(jnp.int32), 0, G - 1)
    cell = gx * G + gy
    bb = lax.broadcasted_iota(jnp.int32, (B, N, N), 0)
    ii = lax.broadcasted_iota(jnp.int32, (B, N, N), 1)
    jj = lax.broadcasted_iota(jnp.int32, (B, N, N), 2)
    flat = cell * ROWS_PAD + bb * N + jj
    o_ref[...] = jnp.where(ii == jj, ZERO_ROW, flat) * 0


def _sc_body(u_hbm, idx_hbm, bias_hbm, out_hbm,
             idx_v, rows0, rows1, acc_v, bias_v, sem0, sem1):
    c = lax.axis_index("c")
    s = lax.axis_index("s")
    wid = s * 2 + c
    pltpu.sync_copy(bias_hbm, bias_v)
    pltpu.sync_copy(idx_hbm.at[pl.ds(wid * NT, NT)], idx_v)
    pltpu.async_copy(u_hbm.at[idx_v.at[0]], rows0, sem0)
    bias_regs = [bias_v[pl.ds(v * 16, 16)] for v in range(D // 16)]

    def reduce_store(buf, t):
        for q in range(GROWS // N):
            for v in range(D // 16):
                sl = pl.ds(v * 16, 16)
                acc = bias_regs[v]
                acc = acc + buf[q * N, sl]
                acc_v[q, sl] = acc
        pltpu.sync_copy(acc_v, out_hbm.at[pl.ds(wid * CHUNK + t * 4, 4)])

    def loop(k, carry):
        t0 = 2 * k
        t1 = t0 + 1
        pltpu.async_copy(u_hbm.at[idx_v.at[t1]], rows1, sem1)
        pltpu.make_async_copy(u_hbm.at[idx_v.at[t0]], rows0, sem0).wait()
        reduce_store(rows0, t0)

        @pl.when(k < NT // 2 - 1)
        def _():
            pltpu.async_copy(u_hbm.at[idx_v.at[t0 + 2]], rows0, sem0)

        pltpu.make_async_copy(u_hbm.at[idx_v.at[t1]], rows1, sem1).wait()
        reduce_store(rows1, t1)
        return carry

    lax.fori_loop(0, NT // 2, loop, 0)


@functools.cache
def _sc_gather_reduce():
    return functools.partial(
        pl.kernel,
        out_type=jax.ShapeDtypeStruct((ROWS, D), jnp.float32),
        mesh=plsc.VectorSubcoreMesh(core_axis_name="c", subcore_axis_name="s"),
        scratch_types=[
            pltpu.VMEM((NT, GROWS), jnp.int32),
            pltpu.VMEM((GROWS, D), jnp.float32),
            pltpu.VMEM((GROWS, D), jnp.float32),
            pltpu.VMEM((4, D), jnp.float32),
            pltpu.VMEM((D,), jnp.float32),
            pltpu.SemaphoreType.DMA,
            pltpu.SemaphoreType.DMA,
        ],
    )(_sc_body)


def kernel(hidden_states, positions, W, b):
    h_pad = jnp.pad(hidden_states.reshape(ROWS, D),
                    ((0, ROWS_PAD - ROWS), (0, 0)))
    px = positions[:, :, 0]
    py = positions[:, :, 1]

    u = pl.pallas_call(
        _mm_body,
        grid=(GG // CELLS_PER_BLK,),
        in_specs=[
            pl.BlockSpec((ROWS_PAD, D), lambda j: (0, 0)),
            pl.BlockSpec((D, CELLS_PER_BLK * D), lambda j: (0, j)),
        ],
        out_specs=pl.BlockSpec((CELLS_PER_BLK * ROWS_PAD, D),
                               lambda j: (j, 0)),
        out_shape=jax.ShapeDtypeStruct((GG * ROWS_PAD, D), jnp.float32),
    )(h_pad, W)

    idx = pl.pallas_call(
        _idx_body,
        out_shape=jax.ShapeDtypeStruct((B, N, N), jnp.int32),
    )(px, py)

    pooled = _sc_gather_reduce()(u, idx.reshape(ROWS * N // GROWS, GROWS), b)
    return pooled.reshape(B, N, D)


# speedup vs baseline: 15.6151x; 15.6151x over previous
import functools

import jax
import jax.numpy as jnp
from jax import lax
from jax.experimental import pallas as pl
from jax.experimental.pallas import tpu as pltpu
from jax.experimental.pallas import tpu_sc as plsc

B, N, D = 64, 32, 128
G = 8
GG = G * G
NH = 4.0
CELL = NH / G

ROWS = B * N
ROWS_PAD = ROWS + 8
ZERO_ROW = ROWS
NUM_WORKERS = 32
CHUNK = ROWS // NUM_WORKERS
GROWS = 128
NT = CHUNK * N // GROWS
NBUF = 4
CELLS_PER_BLK = 4


def _mm_body(h_ref, w_ref, o_ref):
    h = h_ref[...]
    for c in range(CELLS_PER_BLK):
        wblk = w_ref[:, c * D:(c + 1) * D]
        o_ref[c * ROWS_PAD:(c + 1) * ROWS_PAD, :] = lax.dot_general(
            h, wblk, (((1,), (1,)), ((), ())),
            preferred_element_type=jnp.float32)


def _idx_body(px_ref, py_ref, o_ref):
    px = px_ref[...]
    py = py_ref[...]
    rx = px[:, None, :] - px[:, :, None]
    ry = py[:, None, :] - py[:, :, None]
    gx = jnp.clip(((rx + NH / 2.0) / CELL).astype(jnp.int32), 0, G - 1)
    gy = jnp.clip(((ry + NH / 2.0) / CELL).astype(jnp.int32), 0, G - 1)
    cell = gx * G + gy
    bb = lax.broadcasted_iota(jnp.int32, (B, N, N), 0)
    ii = lax.broadcasted_iota(jnp.int32, (B, N, N), 1)
    jj = lax.broadcasted_iota(jnp.int32, (B, N, N), 2)
    flat = cell * ROWS_PAD + bb * N + jj
    o_ref[...] = jnp.where(ii == jj, ZERO_ROW, flat)


def _sc_body(u_hbm, idx_hbm, bias_hbm, out_hbm,
             idx_v, rows0, rows1, rows2, rows3, acc_v, bias_v,
             sem0, sem1, sem2, sem3):
    c = lax.axis_index("c")
    s = lax.axis_index("s")
    wid = s * 2 + c
    bufs = (rows0, rows1, rows2, rows3)
    sems = (sem0, sem1, sem2, sem3)
    pltpu.sync_copy(bias_hbm, bias_v)
    pltpu.sync_copy(idx_hbm.at[pl.ds(wid * NT, NT)], idx_v)
    for t in range(NBUF - 1):
        pltpu.async_copy(u_hbm.at[idx_v.at[t]], bufs[t], sems[t])
    bias_regs = [bias_v[pl.ds(v * 16, 16)] for v in range(D // 16)]

    def reduce_store(buf, t):
        for q in range(GROWS // N):
            for v in range(D // 16):
                sl = pl.ds(v * 16, 16)
                acc = bias_regs[v]
                for r in range(N):
                    acc = acc + buf[q * N + r, sl]
                acc_v[q, sl] = acc
        pltpu.sync_copy(acc_v, out_hbm.at[pl.ds(wid * CHUNK + t * 4, 4)])

    def loop(k, carry):
        for m in range(NBUF):
            t = NBUF * k + m
            pltpu.make_async_copy(u_hbm.at[idx_v.at[t]], bufs[m],
                                  sems[m]).wait()
            reduce_store(bufs[m], t)
            nxt = t + NBUF - 1

            @pl.when(nxt < NT)
            def _():
                pltpu.async_copy(u_hbm.at[idx_v.at[nxt]],
                                 bufs[(m + NBUF - 1) % NBUF],
                                 sems[(m + NBUF - 1) % NBUF])
        return carry

    lax.fori_loop(0, NT // NBUF, loop, 0)


@functools.cache
def _sc_gather_reduce():
    return functools.partial(
        pl.kernel,
        out_type=jax.ShapeDtypeStruct((ROWS, D), jnp.float32),
        mesh=plsc.VectorSubcoreMesh(core_axis_name="c", subcore_axis_name="s"),
        scratch_types=[
            pltpu.VMEM((NT, GROWS), jnp.int32),
            pltpu.VMEM((GROWS, D), jnp.float32),
            pltpu.VMEM((GROWS, D), jnp.float32),
            pltpu.VMEM((GROWS, D), jnp.float32),
            pltpu.VMEM((GROWS, D), jnp.float32),
            pltpu.VMEM((4, D), jnp.float32),
            pltpu.VMEM((D,), jnp.float32),
            pltpu.SemaphoreType.DMA,
            pltpu.SemaphoreType.DMA,
            pltpu.SemaphoreType.DMA,
            pltpu.SemaphoreType.DMA,
        ],
    )(_sc_body)


def kernel(hidden_states, positions, W, b):
    h_pad = jnp.pad(hidden_states.reshape(ROWS, D),
                    ((0, ROWS_PAD - ROWS), (0, 0)))
    px = positions[:, :, 0]
    py = positions[:, :, 1]

    u = pl.pallas_call(
        _mm_body,
        grid=(GG // CELLS_PER_BLK,),
        in_specs=[
            pl.BlockSpec((ROWS_PAD, D), lambda j: (0, 0)),
            pl.BlockSpec((D, CELLS_PER_BLK * D), lambda j: (0, j)),
        ],
        out_specs=pl.BlockSpec((CELLS_PER_BLK * ROWS_PAD, D),
                               lambda j: (j, 0)),
        out_shape=jax.ShapeDtypeStruct((GG * ROWS_PAD, D), jnp.float32),
    )(h_pad, W)

    idx = pl.pallas_call(
        _idx_body,
        out_shape=jax.ShapeDtypeStruct((B, N, N), jnp.int32),
    )(px, py)

    pooled = _sc_gather_reduce()(u, idx.reshape(ROWS * N // GROWS, GROWS), b)
    return pooled.reshape(B, N, D)


# speedup vs baseline: 24.4662x; 1.5668x over previous
import functools

import jax
import jax.numpy as jnp
from jax import lax
from jax.experimental import pallas as pl
from jax.experimental.pallas import tpu as pltpu
from jax.experimental.pallas import tpu_sc as plsc

B, N, D = 64, 32, 128
G = 8
GG = G * G
NH = 4.0
CELL = NH / G

S_SC = 32
B_TC = B - S_SC
SC_PER_BLK = 4

ROWS_SC = S_SC * N
ROWS_PAD = ROWS_SC + 8
ZERO_ROW = ROWS_SC
NUM_WORKERS = 32
CHUNK = ROWS_SC // NUM_WORKERS
GROWS = 128
NT = CHUNK * N // GROWS
NBUF = 2
CELLS_PER_BLK = 4


def _cells(px, py):
    rx = px[:, None, :] - px[:, :, None]
    ry = py[:, None, :] - py[:, :, None]
    gx = jnp.clip(((rx + NH / 2.0) / CELL).astype(jnp.int32), 0, G - 1)
    gy = jnp.clip(((ry + NH / 2.0) / CELL).astype(jnp.int32), 0, G - 1)
    return gx * G + gy


def _mm_body(h_ref, w_ref, o_ref):
    h = h_ref[...]
    for c in range(CELLS_PER_BLK):
        wblk = w_ref[:, c * D:(c + 1) * D]
        o_ref[c * ROWS_PAD:(c + 1) * ROWS_PAD, :] = lax.dot_general(
            h, wblk, (((1,), (1,)), ((), ())),
            preferred_element_type=jnp.float32)


def _idx_body(px_ref, py_ref, o_ref):
    cell = _cells(px_ref[...], py_ref[...])
    bb = lax.broadcasted_iota(jnp.int32, (S_SC, N, N), 0)
    ii = lax.broadcasted_iota(jnp.int32, (S_SC, N, N), 1)
    jj = lax.broadcasted_iota(jnp.int32, (S_SC, N, N), 2)
    flat = cell * ROWS_PAD + bb * N + jj
    o_ref[...] = jnp.where(ii == jj, ZERO_ROW, flat)


def _tc_body(h_ref, px_ref, py_ref, w_ref, b_ref, o_ref):
    cell = _cells(px_ref[0], py_ref[0])
    gfs = []
    for sc in range(SC_PER_BLK):
        c3 = cell[sc][:, None, :]
        g3 = lax.broadcasted_iota(jnp.int32, (N, GG, N), 1)
        ii = lax.broadcasted_iota(jnp.int32, (N, GG, N), 0)
        jj = lax.broadcasted_iota(jnp.int32, (N, GG, N), 2)
        m = ((c3 == g3) & (ii != jj)).astype(jnp.float32)
        gf = lax.dot_general(m.reshape(N * GG, N), h_ref[sc],
                             (((1,), (0,)), ((), ())),
                             preferred_element_type=jnp.float32)
        gfs.append(gf)
    gfa = jnp.concatenate(gfs, axis=0).reshape(SC_PER_BLK, N, GG, D)
    acc = jnp.broadcast_to(b_ref[...].reshape(1, D), (SC_PER_BLK * N, D))
    for g in range(GG):
        xg = gfa[:, :, g, :].reshape(SC_PER_BLK * N, D)
        wg = w_ref[:, g * D:(g + 1) * D]
        acc = acc + lax.dot_general(xg, wg, (((1,), (1,)), ((), ())),
                                    preferred_element_type=jnp.float32)
    o_ref[...] = acc.reshape(SC_PER_BLK, N, D)


def _sc_body(u_hbm, idx_hbm, bias_hbm, out_hbm,
             idx_v, rows0, rows1, acc_v, bias_v, sem0, sem1):
    c = lax.axis_index("c")
    s = lax.axis_index("s")
    wid = s * 2 + c
    pltpu.sync_copy(bias_hbm, bias_v)
    pltpu.sync_copy(idx_hbm.at[pl.ds(wid * NT, NT)], idx_v)
    pltpu.async_copy(u_hbm.at[idx_v.at[0]], rows0, sem0)
    bias_regs = [bias_v[pl.ds(v * 16, 16)] for v in range(D // 16)]

    def reduce_store(buf, t):
        for q in range(GROWS // N):
            for v in range(D // 16):
                sl = pl.ds(v * 16, 16)
                acc = bias_regs[v]
                for r in range(N):
                    acc = acc + buf[q * N + r, sl]
                acc_v[q, sl] = acc
        pltpu.sync_copy(acc_v, out_hbm.at[pl.ds(wid * CHUNK + t * 4, 4)])

    def loop(k, carry):
        t0 = 2 * k
        t1 = t0 + 1
        pltpu.async_copy(u_hbm.at[idx_v.at[t1]], rows1, sem1)
        pltpu.make_async_copy(u_hbm.at[idx_v.at[t0]], rows0, sem0).wait()
        reduce_store(rows0, t0)

        @pl.when(k < NT // 2 - 1)
        def _():
            pltpu.async_copy(u_hbm.at[idx_v.at[t0 + 2]], rows0, sem0)

        pltpu.make_async_copy(u_hbm.at[idx_v.at[t1]], rows1, sem1).wait()
        reduce_store(rows1, t1)
        return carry

    lax.fori_loop(0, NT // 2, loop, 0)


@functools.cache
def _sc_gather_reduce():
    return functools.partial(
        pl.kernel,
        out_type=jax.ShapeDtypeStruct((ROWS_SC, D), jnp.float32),
        mesh=plsc.VectorSubcoreMesh(core_axis_name="c", subcore_axis_name="s"),
        scratch_types=[
            pltpu.VMEM((NT, GROWS), jnp.int32),
            pltpu.VMEM((GROWS, D), jnp.float32),
            pltpu.VMEM((GROWS, D), jnp.float32),
            pltpu.VMEM((4, D), jnp.float32),
            pltpu.VMEM((D,), jnp.float32),
            pltpu.SemaphoreType.DMA,
            pltpu.SemaphoreType.DMA,
        ],
    )(_sc_body)


def kernel(hidden_states, positions, W, b):
    px = positions[:, :, 0]
    py = positions[:, :, 1]
    h_pad = jnp.pad(hidden_states[:S_SC].reshape(ROWS_SC, D),
                    ((0, ROWS_PAD - ROWS_SC), (0, 0)))

    u = pl.pallas_call(
        _mm_body,
        grid=(GG // CELLS_PER_BLK,),
        in_specs=[
            pl.BlockSpec((ROWS_PAD, D), lambda j: (0, 0)),
            pl.BlockSpec((D, CELLS_PER_BLK * D), lambda j: (0, j)),
        ],
        out_specs=pl.BlockSpec((CELLS_PER_BLK * ROWS_PAD, D),
                               lambda j: (j, 0)),
        out_shape=jax.ShapeDtypeStruct((GG * ROWS_PAD, D), jnp.float32),
    )(h_pad, W)

    idx = pl.pallas_call(
        _idx_body,
        out_shape=jax.ShapeDtypeStruct((S_SC, N, N), jnp.int32),
    )(px[:S_SC], py[:S_SC])

    sc_out = _sc_gather_reduce()(u, idx.reshape(ROWS_SC * N // GROWS, GROWS),
                                 b)

    tc_out = pl.pallas_call(
        _tc_body,
        grid=(B_TC // SC_PER_BLK,),
        in_specs=[
            pl.BlockSpec((SC_PER_BLK, N, D), lambda j: (j, 0, 0)),
            pl.BlockSpec((1, SC_PER_BLK, N), lambda j: (j, 0, 0)),
            pl.BlockSpec((1, SC_PER_BLK, N), lambda j: (j, 0, 0)),
            pl.BlockSpec((D, GG * D), lambda j: (0, 0)),
            pl.BlockSpec((1, D), lambda j: (0, 0)),
        ],
        out_specs=pl.BlockSpec((SC_PER_BLK, N, D), lambda j: (j, 0, 0)),
        out_shape=jax.ShapeDtypeStruct((B_TC, N, D), jnp.float32),
    )(hidden_states[S_SC:],
      px[S_SC:].reshape(B_TC // SC_PER_BLK, SC_PER_BLK, N),
      py[S_SC:].reshape(B_TC // SC_PER_BLK, SC_PER_BLK, N), W,
      b.reshape(1, D))

    return jnp.concatenate([sc_out.reshape(S_SC, N, D), tc_out], axis=0)
